# baseline (speedup 1.0000x reference)
import jax
import jax.numpy as jnp
from jax.experimental import pallas as pl
from jax.experimental.pallas import tpu as pltpu

LAMBDA_EXIST = 1.0
LAMBDA_X0 = 1.0
LAMBDA_CNT = 0.1
GAMMA = 2.0
ALPHA = 0.75

_INF = float("inf")
_BIGF = 1e30


def _loss_kernel(pt0, pt1, ep0, ep1, el, p00, p01, p0g0_ref, p0g1_ref,
                 mkf, abar,
                 out_ref, x00, x01, pu_r, aacc0_r, aacc1_r, rank_r):
    f32 = jnp.float32
    B, N = el.shape
    M = p00.shape[1]

    ab = abar[...]
    sa = jnp.sqrt(ab + 1e-6)
    so = jnp.sqrt(jnp.clip(1.0 - ab, 0.0, None))
    lo = -1.0 + 0.001
    hi = 1.0 - 0.001
    x00[...] = jnp.clip((pt0[...] - so * ep0[...]) / sa, lo, hi)
    x01[...] = jnp.clip((pt1[...] - so * ep1[...]) / sa, lo, hi)
    negprob = -(1.0 / (1.0 + jnp.exp(-el[...])))

    iotaN = jax.lax.broadcasted_iota(jnp.int32, (B, N), 1).astype(f32)
    iotaM = jax.lax.broadcasted_iota(jnp.int32, (B, M), 1).astype(f32)
    mk = mkf[...]
    p00v = p00[...]
    p01v = p01[...]
    x00v = x00[...]
    x01v = x01[...]

    pu_r[...] = negprob
    aacc0_r[...] = jnp.zeros((B, N), f32)
    aacc1_r[...] = jnp.zeros((B, N), f32)

    CU = 16
    NG = M // CU
    x003 = x00v[:, None, :]
    x013 = x01v[:, None, :]
    np3 = negprob[:, None, :]
    lane_grp = jax.lax.broadcasted_iota(jnp.int32, (B, M), 1) // CU

    def cm_body(g, colmin):
        pg0 = p0g0_ref[pl.ds(g, 1)][0]
        pg1 = p0g1_ref[pl.ds(g, 1)][0]
        c3 = (jnp.abs(x003 - pg0) + jnp.abs(x013 - pg1)) + np3
        cm16 = jnp.min(c3, axis=2)
        tiled = jnp.concatenate([cm16] * NG, axis=1)
        return jnp.where(lane_grp == g, tiled, colmin)

    colmin = jax.lax.fori_loop(0, NG, cm_body, jnp.zeros((B, M), f32))
    colmin = jnp.where(mk > 0.0, colmin, _INF)

    iota_sub = jax.lax.broadcasted_iota(jnp.int32, (M, M), 0)
    iota_lan = jax.lax.broadcasted_iota(jnp.int32, (M, M), 1)
    diag = iota_sub == iota_lan
    klj = iota_lan < iota_sub
    cm_r = rank_r
    cm_r[...] = colmin

    def rank_body(b, acc):
        cm_row = cm_r[pl.ds(b, 1), :]
        cm_col = jnp.sum(
            jnp.where(diag, jnp.broadcast_to(cm_row, (M, M)), 0.0),
            axis=1, keepdims=True)
        less = cm_row < cm_col
        eqix = (cm_row == cm_col) & klj
        rcol = jnp.sum(jnp.where(less | eqix, 1.0, 0.0), axis=1, keepdims=True)
        rrow = jnp.sum(
            jnp.where(diag, jnp.broadcast_to(rcol, (M, M)), 0.0),
            axis=0, keepdims=True)
        bsel = jax.lax.broadcasted_iota(jnp.int32, (B, M), 0) == b
        return jnp.where(bsel, jnp.broadcast_to(rrow, (B, M)), acc)

    rank = jax.lax.fori_loop(0, B, rank_body, jnp.zeros((B, M), f32))
    rank_r[...] = rank

    GU = 8
    rankv = rank_r[...]
    rank3 = jnp.concatenate([rankv, rankv, rankv], axis=0)
    tbl3 = jnp.concatenate([mk, p00v, p01v], axis=0)

    def gather_aj(t):
        tf = jnp.asarray(t).astype(f32)
        g = jnp.sum(jnp.where(rank3 == tf, tbl3, 0.0), axis=1, keepdims=True)
        return g[0:B], g[B:2 * B], g[2 * B:3 * B]

    def g_body(q, _):
        t0 = GU * q
        pu = pu_r[...]
        ohs = []
        for k in range(GU):
            vm, a0, a1 = gather_aj(t0 + k)
            c = (jnp.abs(x00v - a0) + jnp.abs(x01v - a1)) + pu
            m = jnp.min(c, axis=1, keepdims=True)
            i_f = jnp.min(jnp.where(c == m, iotaN, f32(N)),
                          axis=1, keepdims=True)
            onehot = iotaN == i_f
            upd = jnp.where(vm > 0.0, _INF, _BIGF)
            pu = jnp.where(onehot, upd, pu)
            ohs.append((onehot, a0, a1))
        pu_r[...] = pu
        aacc0 = aacc0_r[...]
        aacc1 = aacc1_r[...]
        for onehot, a0, a1 in ohs:
            aacc0 = jnp.where(onehot, a0, aacc0)
            aacc1 = jnp.where(onehot, a1, aacc1)
        aacc0_r[...] = aacc0
        aacc1_r[...] = aacc1
        return 0

    jax.lax.fori_loop(0, M // GU, g_body, 0)

    y = jnp.where(pu_r[...] == _INF, 1.0, 0.0)
    d0 = jnp.abs(x00v - aacc0_r[...])
    d1 = jnp.abs(x01v - aacc1_r[...])
    sl0 = jnp.where(d0 < 1.0, 0.5 * d0 * d0, d0 - 0.5)
    sl1 = jnp.where(d1 < 1.0, 0.5 * d1 * d1, d1 - 0.5)
    slsum = jnp.sum((sl0 + sl1) * y)

    x = el[...]
    ce = jnp.clip(x, 0.0, None) - x * y + jnp.log1p(jnp.exp(-jnp.abs(x)))
    p = -negprob
    pt = jnp.clip(jnp.where(y == 1.0, p, 1.0 - p), 1e-6, 1.0 - 1e-6)
    at = jnp.where(y == 1.0, ALPHA, 1.0 - ALPHA)
    ompt = 1.0 - pt
    L_exist = jnp.sum(at * (ompt * ompt) * ce) / f32(B * N)
    cnt = jnp.sum(mk)
    L_x0 = jnp.where(cnt > 0.0, slsum / jnp.maximum(cnt * 2.0, 1.0), 0.0)
    pred_cnt = jnp.sum(p, axis=1, keepdims=True)
    gt_cnt = jnp.sum(mk, axis=1, keepdims=True)
    L_cnt = jnp.sum(jnp.abs(pred_cnt - gt_cnt)) / f32(B)
    loss = LAMBDA_X0 * L_x0 + LAMBDA_EXIST * L_exist + LAMBDA_CNT * L_cnt

    out_ref[0:1, :] = jnp.broadcast_to(loss, (1, 128))
    out_ref[1:2, :] = jnp.broadcast_to(L_exist, (1, 128))
    out_ref[2:3, :] = jnp.broadcast_to(L_x0, (1, 128))
    out_ref[3:4, :] = jnp.broadcast_to(L_cnt, (1, 128))


def kernel(p_t, p0, mask, abar_t, eps_pred, exist_logit):
    B, N = exist_logit.shape
    M = p0.shape[1]
    f32 = jnp.float32
    pt0 = p_t[:, :, 0]
    pt1 = p_t[:, :, 1]
    ep0 = eps_pred[:, :, 0]
    ep1 = eps_pred[:, :, 1]
    p00 = p0[:, :, 0]
    p01 = p0[:, :, 1]
    mkf = mask.astype(f32)
    ab = abar_t[:, None]
    CU = 16
    p0g0 = p00.reshape(B, M // CU, CU).transpose(1, 0, 2)[..., None]
    p0g1 = p01.reshape(B, M // CU, CU).transpose(1, 0, 2)[..., None]

    out = pl.pallas_call(
        _loss_kernel,
        out_shape=jax.ShapeDtypeStruct((4, 128), f32),
        scratch_shapes=[
            pltpu.VMEM((B, N), f32),
            pltpu.VMEM((B, N), f32),
            pltpu.VMEM((B, N), f32),
            pltpu.VMEM((B, N), f32),
            pltpu.VMEM((B, N), f32),
            pltpu.VMEM((B, M), f32),
        ],
    )(pt0, pt1, ep0, ep1, exist_logit, p00, p01, p0g0, p0g1, mkf, ab)

    return (out[0, 0], out[1, 0], out[2, 0], out[3, 0])


# speedup vs baseline: 1.1623x; 1.1623x over previous
import jax
import jax.numpy as jnp
from jax.experimental import pallas as pl
from jax.experimental.pallas import tpu as pltpu

LAMBDA_EXIST = 1.0
LAMBDA_X0 = 1.0
LAMBDA_CNT = 0.1
GAMMA = 2.0
ALPHA = 0.75

_INF = float("inf")
_BIGF = 1e30


def _loss_kernel(pt0, pt1, ep0, ep1, el, p00, p01, mkf, abar,
                 out_ref, x00, x01, pu_r, aacc0_r, aacc1_r, rank_r):
    f32 = jnp.float32
    B, N = el.shape
    M = p00.shape[1]

    ab = abar[...]
    sa = jnp.sqrt(ab + 1e-6)
    so = jnp.sqrt(jnp.clip(1.0 - ab, 0.0, None))
    lo = -1.0 + 0.001
    hi = 1.0 - 0.001
    x00[...] = jnp.clip((pt0[...] - so * ep0[...]) / sa, lo, hi)
    x01[...] = jnp.clip((pt1[...] - so * ep1[...]) / sa, lo, hi)
    negprob = -(1.0 / (1.0 + jnp.exp(-el[...])))

    iotaN = jax.lax.broadcasted_iota(jnp.int32, (B, N), 1).astype(f32)
    iotaM = jax.lax.broadcasted_iota(jnp.int32, (B, M), 1).astype(f32)
    mk = mkf[...]
    p00v = p00[...]
    p01v = p01[...]
    x00v = x00[...]
    x01v = x01[...]

    pu_r[...] = negprob
    aacc0_r[...] = jnp.zeros((B, N), f32)
    aacc1_r[...] = jnp.zeros((B, N), f32)

    def cm_one(j, colmin):
        jf = j.astype(f32)
        sel = iotaM == jf
        a0 = jnp.sum(jnp.where(sel, p00v, 0.0), axis=1, keepdims=True)
        a1 = jnp.sum(jnp.where(sel, p01v, 0.0), axis=1, keepdims=True)
        c = (jnp.abs(x00v - a0) + jnp.abs(x01v - a1)) + negprob
        cmj = jnp.min(c, axis=1, keepdims=True)
        return jnp.where(sel, cmj, colmin)

    CU = 16

    def cm_body(jj, colmin):
        for k in range(CU):
            colmin = cm_one(CU * jj + k, colmin)
        return colmin

    colmin = jax.lax.fori_loop(0, M // CU, cm_body, jnp.zeros((B, M), f32))
    colmin = jnp.where(mk > 0.0, colmin, _INF)

    iota_sub = jax.lax.broadcasted_iota(jnp.int32, (M, M), 0)
    iota_lan = jax.lax.broadcasted_iota(jnp.int32, (M, M), 1)
    diag = iota_sub == iota_lan
    klj = iota_lan < iota_sub
    cm_r = rank_r
    cm_r[...] = colmin

    def rank_body(b, acc):
        cm_row = cm_r[pl.ds(b, 1), :]
        cm_col = jnp.sum(
            jnp.where(diag, jnp.broadcast_to(cm_row, (M, M)), 0.0),
            axis=1, keepdims=True)
        less = cm_row < cm_col
        eqix = (cm_row == cm_col) & klj
        rcol = jnp.sum(jnp.where(less | eqix, 1.0, 0.0), axis=1, keepdims=True)
        rrow = jnp.sum(
            jnp.where(diag, jnp.broadcast_to(rcol, (M, M)), 0.0),
            axis=0, keepdims=True)
        bsel = jax.lax.broadcasted_iota(jnp.int32, (B, M), 0) == b
        return jnp.where(bsel, jnp.broadcast_to(rrow, (B, M)), acc)

    rank = jax.lax.fori_loop(0, B, rank_body, jnp.zeros((B, M), f32))
    rank_r[...] = rank

    GU = 8
    rankv = rank_r[...]
    rank3 = jnp.concatenate([rankv, rankv, rankv], axis=0)
    tbl3 = jnp.concatenate([mk, p00v, p01v], axis=0)

    def gather_aj(t):
        tf = jnp.asarray(t).astype(f32)
        g = jnp.sum(jnp.where(rank3 == tf, tbl3, 0.0), axis=1, keepdims=True)
        return g[0:B], g[B:2 * B], g[2 * B:3 * B]

    def g_body(q, _):
        t0 = GU * q
        pu = pu_r[...]
        ohs = []
        for k in range(GU):
            vm, a0, a1 = gather_aj(t0 + k)
            c = (jnp.abs(x00v - a0) + jnp.abs(x01v - a1)) + pu
            m = jnp.min(c, axis=1, keepdims=True)
            i_f = jnp.min(jnp.where(c == m, iotaN, f32(N)),
                          axis=1, keepdims=True)
            onehot = iotaN == i_f
            upd = jnp.where(vm > 0.0, _INF, _BIGF)
            pu = jnp.where(onehot, upd, pu)
            ohs.append((onehot, a0, a1))
        pu_r[...] = pu
        aacc0 = aacc0_r[...]
        aacc1 = aacc1_r[...]
        for onehot, a0, a1 in ohs:
            aacc0 = jnp.where(onehot, a0, aacc0)
            aacc1 = jnp.where(onehot, a1, aacc1)
        aacc0_r[...] = aacc0
        aacc1_r[...] = aacc1
        return 0

    jax.lax.fori_loop(0, M // GU, g_body, 0)

    y = jnp.where(pu_r[...] == _INF, 1.0, 0.0)
    d0 = jnp.abs(x00v - aacc0_r[...])
    d1 = jnp.abs(x01v - aacc1_r[...])
    sl0 = jnp.where(d0 < 1.0, 0.5 * d0 * d0, d0 - 0.5)
    sl1 = jnp.where(d1 < 1.0, 0.5 * d1 * d1, d1 - 0.5)
    slsum = jnp.sum((sl0 + sl1) * y)

    x = el[...]
    ce = jnp.clip(x, 0.0, None) - x * y + jnp.log1p(jnp.exp(-jnp.abs(x)))
    p = -negprob
    pt = jnp.clip(jnp.where(y == 1.0, p, 1.0 - p), 1e-6, 1.0 - 1e-6)
    at = jnp.where(y == 1.0, ALPHA, 1.0 - ALPHA)
    ompt = 1.0 - pt
    L_exist = jnp.sum(at * (ompt * ompt) * ce) / f32(B * N)
    cnt = jnp.sum(mk)
    L_x0 = jnp.where(cnt > 0.0, slsum / jnp.maximum(cnt * 2.0, 1.0), 0.0)
    pred_cnt = jnp.sum(p, axis=1, keepdims=True)
    gt_cnt = jnp.sum(mk, axis=1, keepdims=True)
    L_cnt = jnp.sum(jnp.abs(pred_cnt - gt_cnt)) / f32(B)
    loss = LAMBDA_X0 * L_x0 + LAMBDA_EXIST * L_exist + LAMBDA_CNT * L_cnt

    out_ref[0:1, :] = jnp.broadcast_to(loss, (1, 128))
    out_ref[1:2, :] = jnp.broadcast_to(L_exist, (1, 128))
    out_ref[2:3, :] = jnp.broadcast_to(L_x0, (1, 128))
    out_ref[3:4, :] = jnp.broadcast_to(L_cnt, (1, 128))


def kernel(p_t, p0, mask, abar_t, eps_pred, exist_logit):
    B, N = exist_logit.shape
    M = p0.shape[1]
    f32 = jnp.float32
    pt0 = p_t[:, :, 0]
    pt1 = p_t[:, :, 1]
    ep0 = eps_pred[:, :, 0]
    ep1 = eps_pred[:, :, 1]
    p00 = p0[:, :, 0]
    p01 = p0[:, :, 1]
    mkf = mask.astype(f32)
    ab = abar_t[:, None]

    out = pl.pallas_call(
        _loss_kernel,
        out_shape=jax.ShapeDtypeStruct((4, 128), f32),
        scratch_shapes=[
            pltpu.VMEM((B, N), f32),
            pltpu.VMEM((B, N), f32),
            pltpu.VMEM((B, N), f32),
            pltpu.VMEM((B, N), f32),
            pltpu.VMEM((B, N), f32),
            pltpu.VMEM((B, M), f32),
        ],
    )(pt0, pt1, ep0, ep1, exist_logit, p00, p01, mkf, ab)

    return (out[0, 0], out[1, 0], out[2, 0], out[3, 0])


# speedup vs baseline: 1.1743x; 1.0103x over previous
import jax
import jax.numpy as jnp
from jax.experimental import pallas as pl
from jax.experimental.pallas import tpu as pltpu

LAMBDA_EXIST = 1.0
LAMBDA_X0 = 1.0
LAMBDA_CNT = 0.1
GAMMA = 2.0
ALPHA = 0.75

_INF = float("inf")
_BIGF = 1e30


def _loss_kernel(pt0, pt1, ep0, ep1, el, p00, p01, mkf, abar,
                 out_ref, x00, x01, pu_r, aacc0_r, aacc1_r, rank_r):
    f32 = jnp.float32
    B, N = el.shape
    M = p00.shape[1]

    ab = abar[...]
    sa = jnp.sqrt(ab + 1e-6)
    so = jnp.sqrt(jnp.clip(1.0 - ab, 0.0, None))
    lo = -1.0 + 0.001
    hi = 1.0 - 0.001
    x00[...] = jnp.clip((pt0[...] - so * ep0[...]) / sa, lo, hi)
    x01[...] = jnp.clip((pt1[...] - so * ep1[...]) / sa, lo, hi)
    negprob = -(1.0 / (1.0 + jnp.exp(-el[...])))

    iotaN = jax.lax.broadcasted_iota(jnp.int32, (B, N), 1).astype(f32)
    iotaM = jax.lax.broadcasted_iota(jnp.int32, (B, M), 1).astype(f32)
    mk = mkf[...]
    p00v = p00[...]
    p01v = p01[...]
    x00v = x00[...]
    x01v = x01[...]

    pu_r[...] = negprob
    aacc0_r[...] = jnp.zeros((B, N), f32)
    aacc1_r[...] = jnp.zeros((B, N), f32)

    def cm_one(j, colmin):
        jf = j.astype(f32)
        sel = iotaM == jf
        a0 = jnp.sum(jnp.where(sel, p00v, 0.0), axis=1, keepdims=True)
        a1 = jnp.sum(jnp.where(sel, p01v, 0.0), axis=1, keepdims=True)
        c = (jnp.abs(x00v - a0) + jnp.abs(x01v - a1)) + negprob
        cmj = jnp.min(c, axis=1, keepdims=True)
        return jnp.where(sel, cmj, colmin)

    CU = 16

    def cm_body(jj, colmin):
        for k in range(CU):
            colmin = cm_one(CU * jj + k, colmin)
        return colmin

    colmin = jax.lax.fori_loop(0, M // CU, cm_body, jnp.zeros((B, M), f32))
    colmin = jnp.where(mk > 0.0, colmin, _INF)

    iota_sub = jax.lax.broadcasted_iota(jnp.int32, (M, M), 0)
    iota_lan = jax.lax.broadcasted_iota(jnp.int32, (M, M), 1)
    diag = iota_sub == iota_lan
    klj = iota_lan < iota_sub
    cm_r = rank_r
    cm_r[...] = colmin

    def rank_body(b, acc):
        cm_row = cm_r[pl.ds(b, 1), :]
        cm_col = jnp.sum(
            jnp.where(diag, jnp.broadcast_to(cm_row, (M, M)), 0.0),
            axis=1, keepdims=True)
        less = cm_row < cm_col
        eqix = (cm_row == cm_col) & klj
        rcol = jnp.sum(jnp.where(less | eqix, 1.0, 0.0), axis=1, keepdims=True)
        rrow = jnp.sum(
            jnp.where(diag, jnp.broadcast_to(rcol, (M, M)), 0.0),
            axis=0, keepdims=True)
        bsel = jax.lax.broadcasted_iota(jnp.int32, (B, M), 0) == b
        return jnp.where(bsel, jnp.broadcast_to(rrow, (B, M)), acc)

    rank = jax.lax.fori_loop(0, B, rank_body, jnp.zeros((B, M), f32))
    rank_r[...] = rank

    GU = 16
    rankv = rank_r[...]
    rank3 = jnp.concatenate([rankv, rankv, rankv], axis=0)
    tbl3 = jnp.concatenate([mk, p00v, p01v], axis=0)

    def gather_aj(t):
        tf = jnp.asarray(t).astype(f32)
        g = jnp.sum(jnp.where(rank3 == tf, tbl3, 0.0), axis=1, keepdims=True)
        return g[0:B], g[B:2 * B], g[2 * B:3 * B]

    def g_body(q, _):
        t0 = GU * q
        pu = pu_r[...]
        ohs = []
        for k in range(GU):
            vm, a0, a1 = gather_aj(t0 + k)
            c = (jnp.abs(x00v - a0) + jnp.abs(x01v - a1)) + pu
            m = jnp.min(c, axis=1, keepdims=True)
            i_f = jnp.min(jnp.where(c == m, iotaN, f32(N)),
                          axis=1, keepdims=True)
            onehot = iotaN == i_f
            upd = jnp.where(vm > 0.0, _INF, _BIGF)
            pu = jnp.where(onehot, upd, pu)
            ohs.append((onehot, a0, a1))
        pu_r[...] = pu
        aacc0 = aacc0_r[...]
        aacc1 = aacc1_r[...]
        for onehot, a0, a1 in ohs:
            aacc0 = jnp.where(onehot, a0, aacc0)
            aacc1 = jnp.where(onehot, a1, aacc1)
        aacc0_r[...] = aacc0
        aacc1_r[...] = aacc1
        return 0

    jax.lax.fori_loop(0, M // GU, g_body, 0)

    y = jnp.where(pu_r[...] == _INF, 1.0, 0.0)
    d0 = jnp.abs(x00v - aacc0_r[...])
    d1 = jnp.abs(x01v - aacc1_r[...])
    sl0 = jnp.where(d0 < 1.0, 0.5 * d0 * d0, d0 - 0.5)
    sl1 = jnp.where(d1 < 1.0, 0.5 * d1 * d1, d1 - 0.5)
    slsum = jnp.sum((sl0 + sl1) * y)

    x = el[...]
    ce = jnp.clip(x, 0.0, None) - x * y + jnp.log1p(jnp.exp(-jnp.abs(x)))
    p = -negprob
    pt = jnp.clip(jnp.where(y == 1.0, p, 1.0 - p), 1e-6, 1.0 - 1e-6)
    at = jnp.where(y == 1.0, ALPHA, 1.0 - ALPHA)
    ompt = 1.0 - pt
    L_exist = jnp.sum(at * (ompt * ompt) * ce) / f32(B * N)
    cnt = jnp.sum(mk)
    L_x0 = jnp.where(cnt > 0.0, slsum / jnp.maximum(cnt * 2.0, 1.0), 0.0)
    pred_cnt = jnp.sum(p, axis=1, keepdims=True)
    gt_cnt = jnp.sum(mk, axis=1, keepdims=True)
    L_cnt = jnp.sum(jnp.abs(pred_cnt - gt_cnt)) / f32(B)
    loss = LAMBDA_X0 * L_x0 + LAMBDA_EXIST * L_exist + LAMBDA_CNT * L_cnt

    out_ref[0:1, :] = jnp.broadcast_to(loss, (1, 128))
    out_ref[1:2, :] = jnp.broadcast_to(L_exist, (1, 128))
    out_ref[2:3, :] = jnp.broadcast_to(L_x0, (1, 128))
    out_ref[3:4, :] = jnp.broadcast_to(L_cnt, (1, 128))


def kernel(p_t, p0, mask, abar_t, eps_pred, exist_logit):
    B, N = exist_logit.shape
    M = p0.shape[1]
    f32 = jnp.float32
    pt0 = p_t[:, :, 0]
    pt1 = p_t[:, :, 1]
    ep0 = eps_pred[:, :, 0]
    ep1 = eps_pred[:, :, 1]
    p00 = p0[:, :, 0]
    p01 = p0[:, :, 1]
    mkf = mask.astype(f32)
    ab = abar_t[:, None]

    out = pl.pallas_call(
        _loss_kernel,
        out_shape=jax.ShapeDtypeStruct((4, 128), f32),
        scratch_shapes=[
            pltpu.VMEM((B, N), f32),
            pltpu.VMEM((B, N), f32),
            pltpu.VMEM((B, N), f32),
            pltpu.VMEM((B, N), f32),
            pltpu.VMEM((B, N), f32),
            pltpu.VMEM((B, M), f32),
        ],
    )(pt0, pt1, ep0, ep1, exist_logit, p00, p01, mkf, ab)

    return (out[0, 0], out[1, 0], out[2, 0], out[3, 0])
